# Initial kernel scaffold; baseline (speedup 1.0000x reference)
import jax
import jax.numpy as jnp
from jax.experimental import pallas as pl


def kernel(x, weight, bias):
    raise NotImplementedError("write your pallas kernel here")


# speedup vs baseline: 2.2550x; 2.2550x over previous
import functools

import jax
import jax.numpy as jnp
from jax.experimental import pallas as pl

_N_GROUPS = 4
_TOPK_GROUPS = 2
_TOPK = 2
_ROUTE_SCALE = 1.0
_N_EXPERTS = 8
_TILE = 512


def _gate_kernel(x_ref, w_ref, b_ref, wout_ref, iout_ref):
    xb = x_ref[...].astype(jnp.bfloat16)
    wb = w_ref[...].astype(jnp.bfloat16)
    scores = jax.lax.dot_general(
        xb, wb, (((1,), (1,)), ((), ())),
        preferred_element_type=jnp.float32)
    scores = jax.nn.sigmoid(scores) + b_ref[...]

    t = scores.shape[0]
    e = _N_EXPERTS
    e_iota = jax.lax.broadcasted_iota(jnp.int32, (t, e), 1)
    gid = e_iota // (e // _N_GROUPS)

    cols = [scores[:, j:j + 1] for j in range(e)]
    gsums = [cols[2 * g] + cols[2 * g + 1] for g in range(_N_GROUPS)]
    p = jnp.concatenate([gsums[j // 2] for j in range(e)], axis=1)

    neg_inf = jnp.float32(-jnp.inf)
    big = jnp.int32(e)

    m1 = jnp.max(p, axis=1, keepdims=True)
    g1 = jnp.min(jnp.where(p == m1, gid, big), axis=1, keepdims=True)
    p2 = jnp.where(gid == g1, neg_inf, p)
    m2 = jnp.max(p2, axis=1, keepdims=True)
    g2 = jnp.min(jnp.where(p2 == m2, gid, big), axis=1, keepdims=True)

    sel = (gid == g1) | (gid == g2)
    s_masked = jnp.where(sel, scores, jnp.float32(0.0))

    m1e = jnp.max(s_masked, axis=1, keepdims=True)
    i1 = jnp.min(jnp.where(s_masked == m1e, e_iota, big), axis=1, keepdims=True)
    s2 = jnp.where(e_iota == i1, neg_inf, s_masked)
    m2e = jnp.max(s2, axis=1, keepdims=True)
    i2 = jnp.min(jnp.where(s2 == m2e, e_iota, big), axis=1, keepdims=True)

    w1 = jnp.sum(jnp.where(e_iota == i1, scores, 0.0), axis=1, keepdims=True)
    w2 = jnp.sum(jnp.where(e_iota == i2, scores, 0.0), axis=1, keepdims=True)
    denom = w1 + w2
    scale = jnp.float32(_ROUTE_SCALE)
    wout_ref[...] = jnp.concatenate([w1 / denom, w2 / denom], axis=1) * scale
    iout_ref[...] = jnp.concatenate([i1, i2], axis=1)


@jax.jit
def kernel(x, weight, bias):
    tokens, dim = x.shape
    e = weight.shape[0]
    grid = (tokens // _TILE,)
    wout, iout = pl.pallas_call(
        _gate_kernel,
        grid=grid,
        in_specs=[
            pl.BlockSpec((_TILE, dim), lambda i: (i, 0)),
            pl.BlockSpec((e, dim), lambda i: (0, 0)),
            pl.BlockSpec((1, e), lambda i: (0, 0)),
        ],
        out_specs=[
            pl.BlockSpec((_TILE, _TOPK), lambda i: (i, 0)),
            pl.BlockSpec((_TILE, _TOPK), lambda i: (i, 0)),
        ],
        out_shape=[
            jax.ShapeDtypeStruct((tokens, _TOPK), jnp.float32),
            jax.ShapeDtypeStruct((tokens, _TOPK), jnp.int32),
        ],
    )(x, weight, bias.reshape(1, e))
    return wout.astype(x.dtype), iout


# speedup vs baseline: 2.3945x; 1.0619x over previous
import functools

import jax
import jax.numpy as jnp
from jax.experimental import pallas as pl

_N_GROUPS = 4
_TOPK_GROUPS = 2
_TOPK = 2
_ROUTE_SCALE = 1.0
_N_EXPERTS = 8
_TILE = 1024


def _gate_kernel(x_ref, w_ref, b_ref, wout_ref, iout_ref):
    xb = x_ref[...].astype(jnp.bfloat16)
    wb = w_ref[...].astype(jnp.bfloat16)
    scores = jax.lax.dot_general(
        xb, wb, (((1,), (1,)), ((), ())),
        preferred_element_type=jnp.float32)
    scores = jax.nn.sigmoid(scores) + b_ref[...]

    t = scores.shape[0]
    e = _N_EXPERTS
    e_iota = jax.lax.broadcasted_iota(jnp.int32, (t, e), 1)
    gid = e_iota // (e // _N_GROUPS)

    cols = [scores[:, j:j + 1] for j in range(e)]
    gsums = [cols[2 * g] + cols[2 * g + 1] for g in range(_N_GROUPS)]
    p = jnp.concatenate([gsums[j // 2] for j in range(e)], axis=1)

    neg_inf = jnp.float32(-jnp.inf)
    big = jnp.int32(e)

    m1 = jnp.max(p, axis=1, keepdims=True)
    g1 = jnp.min(jnp.where(p == m1, gid, big), axis=1, keepdims=True)
    p2 = jnp.where(gid == g1, neg_inf, p)
    m2 = jnp.max(p2, axis=1, keepdims=True)
    g2 = jnp.min(jnp.where(p2 == m2, gid, big), axis=1, keepdims=True)

    sel = (gid == g1) | (gid == g2)
    s_masked = jnp.where(sel, scores, jnp.float32(0.0))

    m1e = jnp.max(s_masked, axis=1, keepdims=True)
    i1 = jnp.min(jnp.where(s_masked == m1e, e_iota, big), axis=1, keepdims=True)
    s2 = jnp.where(e_iota == i1, neg_inf, s_masked)
    m2e = jnp.max(s2, axis=1, keepdims=True)
    i2 = jnp.min(jnp.where(s2 == m2e, e_iota, big), axis=1, keepdims=True)

    w1 = jnp.sum(jnp.where(e_iota == i1, scores, 0.0), axis=1, keepdims=True)
    w2 = jnp.sum(jnp.where(e_iota == i2, scores, 0.0), axis=1, keepdims=True)
    denom = w1 + w2
    scale = jnp.float32(_ROUTE_SCALE)
    wout_ref[...] = jnp.concatenate([w1 / denom, w2 / denom], axis=1) * scale
    iout_ref[...] = jnp.concatenate([i1, i2], axis=1)


@jax.jit
def kernel(x, weight, bias):
    tokens, dim = x.shape
    e = weight.shape[0]
    grid = (tokens // _TILE,)
    wout, iout = pl.pallas_call(
        _gate_kernel,
        grid=grid,
        in_specs=[
            pl.BlockSpec((_TILE, dim), lambda i: (i, 0)),
            pl.BlockSpec((e, dim), lambda i: (0, 0)),
            pl.BlockSpec((1, e), lambda i: (0, 0)),
        ],
        out_specs=[
            pl.BlockSpec((_TILE, _TOPK), lambda i: (i, 0)),
            pl.BlockSpec((_TILE, _TOPK), lambda i: (i, 0)),
        ],
        out_shape=[
            jax.ShapeDtypeStruct((tokens, _TOPK), jnp.float32),
            jax.ShapeDtypeStruct((tokens, _TOPK), jnp.int32),
        ],
    )(x, weight, bias.reshape(1, e))
    return wout.astype(x.dtype), iout


# speedup vs baseline: 2.6786x; 1.1186x over previous
import functools

import jax
import jax.numpy as jnp
from jax import lax
from jax.experimental import pallas as pl
from jax.experimental.pallas import tpu as pltpu
from jax.experimental.pallas import tpu_sc as plsc

_N_GROUPS = 4
_TOPK_GROUPS = 2
_TOPK = 2
_ROUTE_SCALE = 1.0
_N_EXPERTS = 8
_TILE = 1024

_NC = 2
_NS = 16
_NW = _NC * _NS
_L = 16


def _score_kernel(x_ref, w_ref, b_ref, s_ref):
    xb = x_ref[...].astype(jnp.bfloat16)
    wb = w_ref[...].astype(jnp.bfloat16)
    scores_t = jax.lax.dot_general(
        wb, xb, (((1,), (1,)), ((), ())),
        preferred_element_type=jnp.float32)
    s_ref[...] = jax.nn.sigmoid(scores_t) + b_ref[...]


def _route_kernel(tokens, s_hbm, wout_hbm, iout_hbm,
                  sbuf, w1buf, w2buf, i1buf, i2buf):
    per_w = tokens // _NW
    wid = lax.axis_index("s") * _NC + lax.axis_index("c")
    base = wid * per_w
    pltpu.sync_copy(s_hbm.at[:, pl.ds(base, per_w)], sbuf)

    e = _N_EXPERTS
    neg_inf = jnp.full((_L,), -jnp.inf, jnp.float32)
    fzero = jnp.zeros((_L,), jnp.float32)

    def body(j, carry):
        off = j * _L
        s = [sbuf[k, pl.ds(off, _L)] for k in range(e)]

        p = [s[2 * g] + s[2 * g + 1] for g in range(_N_GROUPS)]

        m1 = jnp.maximum(jnp.maximum(p[0], p[1]), jnp.maximum(p[2], p[3]))
        g1 = jnp.full((_L,), _N_GROUPS - 1, jnp.int32)
        for g in range(_N_GROUPS - 2, -1, -1):
            g1 = jnp.where(p[g] == m1, jnp.full((_L,), g, jnp.int32), g1)
        pm = [jnp.where(g1 == jnp.full((_L,), g, jnp.int32), neg_inf, p[g])
              for g in range(_N_GROUPS)]
        m2 = jnp.maximum(jnp.maximum(pm[0], pm[1]), jnp.maximum(pm[2], pm[3]))
        g2 = jnp.full((_L,), _N_GROUPS - 1, jnp.int32)
        for g in range(_N_GROUPS - 2, -1, -1):
            g2 = jnp.where(pm[g] == m2, jnp.full((_L,), g, jnp.int32), g2)

        sm = []
        for k in range(e):
            gk = jnp.full((_L,), k // (e // _N_GROUPS), jnp.int32)
            sel = (g1 == gk) | (g2 == gk)
            sm.append(jnp.where(sel, s[k], fzero))

        m1e = sm[0]
        for k in range(1, e):
            m1e = jnp.maximum(m1e, sm[k])
        i1 = jnp.full((_L,), e - 1, jnp.int32)
        for k in range(e - 2, -1, -1):
            i1 = jnp.where(sm[k] == m1e, jnp.full((_L,), k, jnp.int32), i1)
        sm2 = [jnp.where(i1 == jnp.full((_L,), k, jnp.int32), neg_inf, sm[k])
               for k in range(e)]
        m2e = sm2[0]
        for k in range(1, e):
            m2e = jnp.maximum(m2e, sm2[k])
        i2 = jnp.full((_L,), e - 1, jnp.int32)
        for k in range(e - 2, -1, -1):
            i2 = jnp.where(sm2[k] == m2e, jnp.full((_L,), k, jnp.int32), i2)

        w1 = s[e - 1]
        w2 = s[e - 1]
        for k in range(e - 2, -1, -1):
            ik = jnp.full((_L,), k, jnp.int32)
            w1 = jnp.where(i1 == ik, s[k], w1)
            w2 = jnp.where(i2 == ik, s[k], w2)
        denom = w1 + w2
        scale = jnp.full((_L,), _ROUTE_SCALE, jnp.float32)
        w1buf[pl.ds(off, _L)] = w1 / denom * scale
        w2buf[pl.ds(off, _L)] = w2 / denom * scale
        i1buf[pl.ds(off, _L)] = i1
        i2buf[pl.ds(off, _L)] = i2
        return carry

    lax.fori_loop(0, per_w // _L, body, 0)
    pltpu.sync_copy(w1buf, wout_hbm.at[0, pl.ds(base, per_w)])
    pltpu.sync_copy(w2buf, wout_hbm.at[1, pl.ds(base, per_w)])
    pltpu.sync_copy(i1buf, iout_hbm.at[0, pl.ds(base, per_w)])
    pltpu.sync_copy(i2buf, iout_hbm.at[1, pl.ds(base, per_w)])


@jax.jit
def kernel(x, weight, bias):
    tokens, dim = x.shape
    e = weight.shape[0]
    scores_t = pl.pallas_call(
        _score_kernel,
        grid=(tokens // _TILE,),
        in_specs=[
            pl.BlockSpec((_TILE, dim), lambda i: (i, 0)),
            pl.BlockSpec((e, dim), lambda i: (0, 0)),
            pl.BlockSpec((e, 1), lambda i: (0, 0)),
        ],
        out_specs=pl.BlockSpec((e, _TILE), lambda i: (0, i)),
        out_shape=jax.ShapeDtypeStruct((e, tokens), jnp.float32),
    )(x, weight, bias.reshape(e, 1))

    per_w = tokens // _NW
    mesh = plsc.VectorSubcoreMesh(core_axis_name="c", subcore_axis_name="s")
    route = pl.kernel(
        functools.partial(_route_kernel, tokens),
        out_type=[
            jax.ShapeDtypeStruct((_TOPK, tokens), jnp.float32),
            jax.ShapeDtypeStruct((_TOPK, tokens), jnp.int32),
        ],
        mesh=mesh,
        scratch_types=[
            pltpu.VMEM((e, per_w), jnp.float32),
            pltpu.VMEM((per_w,), jnp.float32),
            pltpu.VMEM((per_w,), jnp.float32),
            pltpu.VMEM((per_w,), jnp.int32),
            pltpu.VMEM((per_w,), jnp.int32),
        ],
    )
    wout_t, iout_t = route(scores_t)
    return wout_t.T.astype(x.dtype), iout_t.T


# speedup vs baseline: 2.7221x; 1.0163x over previous
import functools

import jax
import jax.numpy as jnp
from jax import lax
from jax.experimental import pallas as pl
from jax.experimental.pallas import tpu as pltpu
from jax.experimental.pallas import tpu_sc as plsc

_N_GROUPS = 4
_TOPK_GROUPS = 2
_TOPK = 2
_ROUTE_SCALE = 1.0
_N_EXPERTS = 8
_TILE = 512

_NC = 2
_NS = 16
_NW = _NC * _NS
_L = 16


def _score_kernel(x_ref, w_ref, b_ref, s_ref):
    xb = x_ref[...].astype(jnp.bfloat16)
    wb = w_ref[...].astype(jnp.bfloat16)
    scores_t = jax.lax.dot_general(
        wb, xb, (((1,), (1,)), ((), ())),
        preferred_element_type=jnp.float32)
    s_ref[...] = jax.nn.sigmoid(scores_t) + b_ref[...]


def _route_kernel(tokens, s_hbm, wout_hbm, iout_hbm,
                  sbuf, w1buf, w2buf, i1buf, i2buf):
    per_w = tokens // _NW
    wid = lax.axis_index("s") * _NC + lax.axis_index("c")
    base = wid * per_w
    pltpu.sync_copy(s_hbm.at[:, pl.ds(base, per_w)], sbuf)

    e = _N_EXPERTS
    neg_inf = jnp.full((_L,), -jnp.inf, jnp.float32)
    fzero = jnp.zeros((_L,), jnp.float32)

    def body(j, carry):
        off = j * _L
        s = [sbuf[k, pl.ds(off, _L)] for k in range(e)]

        p = [s[2 * g] + s[2 * g + 1] for g in range(_N_GROUPS)]

        m1 = jnp.maximum(jnp.maximum(p[0], p[1]), jnp.maximum(p[2], p[3]))
        g1 = jnp.full((_L,), _N_GROUPS - 1, jnp.int32)
        for g in range(_N_GROUPS - 2, -1, -1):
            g1 = jnp.where(p[g] == m1, jnp.full((_L,), g, jnp.int32), g1)
        pm = [jnp.where(g1 == jnp.full((_L,), g, jnp.int32), neg_inf, p[g])
              for g in range(_N_GROUPS)]
        m2 = jnp.maximum(jnp.maximum(pm[0], pm[1]), jnp.maximum(pm[2], pm[3]))
        g2 = jnp.full((_L,), _N_GROUPS - 1, jnp.int32)
        for g in range(_N_GROUPS - 2, -1, -1):
            g2 = jnp.where(pm[g] == m2, jnp.full((_L,), g, jnp.int32), g2)

        sm = []
        for k in range(e):
            gk = jnp.full((_L,), k // (e // _N_GROUPS), jnp.int32)
            sel = (g1 == gk) | (g2 == gk)
            sm.append(jnp.where(sel, s[k], fzero))

        m1e = sm[0]
        for k in range(1, e):
            m1e = jnp.maximum(m1e, sm[k])
        i1 = jnp.full((_L,), e - 1, jnp.int32)
        for k in range(e - 2, -1, -1):
            i1 = jnp.where(sm[k] == m1e, jnp.full((_L,), k, jnp.int32), i1)
        sm2 = [jnp.where(i1 == jnp.full((_L,), k, jnp.int32), neg_inf, sm[k])
               for k in range(e)]
        m2e = sm2[0]
        for k in range(1, e):
            m2e = jnp.maximum(m2e, sm2[k])
        i2 = jnp.full((_L,), e - 1, jnp.int32)
        for k in range(e - 2, -1, -1):
            i2 = jnp.where(sm2[k] == m2e, jnp.full((_L,), k, jnp.int32), i2)

        w1 = s[e - 1]
        w2 = s[e - 1]
        for k in range(e - 2, -1, -1):
            ik = jnp.full((_L,), k, jnp.int32)
            w1 = jnp.where(i1 == ik, s[k], w1)
            w2 = jnp.where(i2 == ik, s[k], w2)
        denom = w1 + w2
        scale = jnp.full((_L,), _ROUTE_SCALE, jnp.float32)
        w1buf[pl.ds(off, _L)] = w1 / denom * scale
        w2buf[pl.ds(off, _L)] = w2 / denom * scale
        i1buf[pl.ds(off, _L)] = i1
        i2buf[pl.ds(off, _L)] = i2
        return carry

    lax.fori_loop(0, per_w // _L, body, 0)
    pltpu.sync_copy(w1buf, wout_hbm.at[0, pl.ds(base, per_w)])
    pltpu.sync_copy(w2buf, wout_hbm.at[1, pl.ds(base, per_w)])
    pltpu.sync_copy(i1buf, iout_hbm.at[0, pl.ds(base, per_w)])
    pltpu.sync_copy(i2buf, iout_hbm.at[1, pl.ds(base, per_w)])


@jax.jit
def kernel(x, weight, bias):
    tokens, dim = x.shape
    e = weight.shape[0]
    scores_t = pl.pallas_call(
        _score_kernel,
        grid=(tokens // _TILE,),
        in_specs=[
            pl.BlockSpec((_TILE, dim), lambda i: (i, 0)),
            pl.BlockSpec((e, dim), lambda i: (0, 0)),
            pl.BlockSpec((e, 1), lambda i: (0, 0)),
        ],
        out_specs=pl.BlockSpec((e, _TILE), lambda i: (0, i)),
        out_shape=jax.ShapeDtypeStruct((e, tokens), jnp.float32),
    )(x, weight, bias.reshape(e, 1))

    per_w = tokens // _NW
    mesh = plsc.VectorSubcoreMesh(core_axis_name="c", subcore_axis_name="s")
    route = pl.kernel(
        functools.partial(_route_kernel, tokens),
        out_type=[
            jax.ShapeDtypeStruct((_TOPK, tokens), jnp.float32),
            jax.ShapeDtypeStruct((_TOPK, tokens), jnp.int32),
        ],
        mesh=mesh,
        scratch_types=[
            pltpu.VMEM((e, per_w), jnp.float32),
            pltpu.VMEM((per_w,), jnp.float32),
            pltpu.VMEM((per_w,), jnp.float32),
            pltpu.VMEM((per_w,), jnp.int32),
            pltpu.VMEM((per_w,), jnp.int32),
        ],
    )
    wout_t, iout_t = route(scores_t)
    return wout_t.T.astype(x.dtype), iout_t.T
